# Initial kernel scaffold; baseline (speedup 1.0000x reference)
import jax
import jax.numpy as jnp
from jax.experimental import pallas as pl


def kernel(y_prime, B, L):
    raise NotImplementedError("write your pallas kernel here")


# speedup vs baseline: 1.6615x; 1.6615x over previous
import functools

import jax
import jax.numpy as jnp
from jax import lax
from jax.experimental import pallas as pl
from jax.experimental.pallas import tpu as pltpu

C_TOTAL = 4096
B_MAX_CONST = 100.0


def _importance_body(y_ref, imp_ref, *, nblocks):
    n = pl.program_id(1)
    part = jnp.sum(jnp.abs(y_ref[0]), axis=0, keepdims=True)[None]

    @pl.when(n == 0)
    def _():
        imp_ref[...] = part

    @pl.when(n > 0)
    def _():
        imp_ref[...] = imp_ref[...] + part


def _mask_body(imp_ref, b_ref, l_ref, mask_ref, k_ref):
    bsz, _, c = imp_ref.shape
    b_vec = b_ref[...]
    l_vec = l_ref[...]
    b_vec = jnp.where(jnp.max(b_vec) > B_MAX_CONST * 1.2, b_vec / 1e6, b_vec)
    l_vec = jnp.where(jnp.max(l_vec) < 1.0, l_vec * 1000.0, l_vec)
    k_f = jnp.round(c * b_vec / (1.0 + l_vec / 500.0))
    k = jnp.clip(k_f.astype(jnp.int32), 1, c)
    k_ref[...] = k

    imp = imp_ref[...][:, 0, :]
    bits = lax.bitcast_convert_type(imp, jnp.int32)

    k2 = k[:, None]

    def ge_count(t):
        return jnp.sum((bits >= t).astype(jnp.int32), axis=1, keepdims=True)

    def step(_, carry):
        lo, hi = carry
        mid = lo + (hi - lo + 1) // 2
        cnt = ge_count(mid)
        go_up = cnt >= k2
        lo = jnp.where(go_up, mid, lo)
        hi = jnp.where(go_up, hi, mid - 1)
        return lo, hi

    lo0 = jnp.zeros((bsz, 1), jnp.int32)
    hi0 = jnp.full((bsz, 1), jnp.int32(0x7F800000))
    lo, hi = lax.fori_loop(0, 31, step, (lo0, hi0))
    t = lo

    gt = bits > t
    eq = bits == t
    need = k2 - jnp.sum(gt.astype(jnp.int32), axis=1, keepdims=True)

    idx = lax.broadcasted_iota(jnp.int32, (bsz, c), 1)

    def eq_count(m):
        return jnp.sum((eq & (idx < m)).astype(jnp.int32), axis=1, keepdims=True)

    def step2(_, carry):
        lo2, hi2 = carry
        mid = (lo2 + hi2) // 2
        enough = eq_count(mid) >= need
        hi2 = jnp.where(enough, mid, hi2)
        lo2 = jnp.where(enough, lo2, mid)
        return lo2, hi2

    lo2, hi2 = lax.fori_loop(
        0, 12, step2, (jnp.zeros((bsz, 1), jnp.int32), jnp.full((bsz, 1), c))
    )
    m_star = hi2

    mask = gt | (eq & (idx < m_star))
    mask_ref[...] = mask.astype(jnp.int32)[:, None, :]


def _apply_body(y_ref, mask_ref, out_ref):
    out_ref[...] = jnp.where(mask_ref[...] != 0, y_ref[...], 0.0)


@jax.jit
def kernel(y_prime, B, L):
    bsz, n_dim, c = y_prime.shape
    nblk = min(512, n_dim)
    nblocks = n_dim // nblk

    importance = pl.pallas_call(
        functools.partial(_importance_body, nblocks=nblocks),
        grid=(bsz, nblocks),
        in_specs=[pl.BlockSpec((1, nblk, c), lambda b, n: (b, n, 0))],
        out_specs=pl.BlockSpec((1, 1, c), lambda b, n: (b, 0, 0)),
        out_shape=jax.ShapeDtypeStruct((bsz, 1, c), jnp.float32),
    )(y_prime)

    mask_i32, k_vec = pl.pallas_call(
        _mask_body,
        out_shape=[
            jax.ShapeDtypeStruct((bsz, 1, c), jnp.int32),
            jax.ShapeDtypeStruct((bsz,), jnp.int32),
        ],
    )(importance, B.reshape(-1).astype(jnp.float32), L.reshape(-1).astype(jnp.float32))

    y_masked = pl.pallas_call(
        _apply_body,
        grid=(bsz, nblocks),
        in_specs=[
            pl.BlockSpec((1, nblk, c), lambda b, n: (b, n, 0)),
            pl.BlockSpec((1, 1, c), lambda b, n: (b, 0, 0)),
        ],
        out_specs=pl.BlockSpec((1, nblk, c), lambda b, n: (b, n, 0)),
        out_shape=jax.ShapeDtypeStruct((bsz, n_dim, c), jnp.float32),
    )(y_prime, mask_i32)

    mask = mask_i32.astype(bool)
    return (y_masked, mask, k_vec)
